# baseline (speedup 1.0000x reference)
import functools

import jax
import jax.numpy as jnp
from jax import lax
from jax.experimental import pallas as pl
from jax.experimental.pallas import tpu as pltpu
from jax.experimental.pallas import tpu_sc as plsc

N = 10000
E = 320000
D = 128
DE = 16
PRW = 16
B = 8
P0 = 256
P1 = 64
P2 = 16
NT0 = 4
NT1 = 4
NT2 = 1

NC = 2
NS = 16
NW = NC * NS
EC = 64
N_CHUNKS = E // EC
CHUNK_ITERS = -(-N_CHUNKS // NW)
ZROWS = 40
TILE_ROWS = 640


def _mp_body(h_hbm, e_hbm, src_hbm, dst_hbm, out_hbm,
             src_v, dst_v, dstS_v, hrow_v, erow_v, agg_sh,
             semL, semG, semS):
    c = lax.axis_index("c")
    s = lax.axis_index("s")
    wid = s * NC + c
    n_copies = jnp.where(s < NS - 1, TILE_ROWS // ZROWS, 10)

    def zrow(i, carry):
        for g in range(8):
            hrow_v[0, i, pl.ds(g * 16, 16)] = jnp.zeros((16,), jnp.float32)
        return carry

    lax.fori_loop(0, ZROWS, zrow, 0)

    def zcp(j, carry):
        pltpu.sync_copy(hrow_v.at[0, pl.ds(0, ZROWS)],
                        agg_sh.at[pl.ds(s * TILE_ROWS + j * ZROWS, ZROWS)])
        return carry

    lax.fori_loop(0, n_copies, zcp, 0)
    plsc.subcore_barrier()

    def fire_loads(b, cid):
        base = cid * EC
        pltpu.async_copy(src_hbm.at[pl.ds(base, EC)], src_v.at[b], semL[b])
        pltpu.async_copy(dst_hbm.at[pl.ds(base, EC)], dst_v.at[b], semL[b])
        pltpu.async_copy(e_hbm.at[pl.ds(cid * (EC // 8), EC // 8)],
                         erow_v.at[b], semL[b])

    for b in range(2):
        fire_loads(b, b * NW + wid)

    def round_for(b, cid):
        @pl.when(cid < N_CHUNKS)
        def _():
            @pl.when(cid >= 2 * NW)
            def _():
                pltpu.make_async_copy(hrow_v.at[b],
                                      agg_sh.at[dstS_v.at[b]], semS[b]).wait()

            base = cid * EC
            pltpu.make_async_copy(src_hbm.at[pl.ds(base, EC)], src_v.at[b],
                                  semL[b]).wait()
            pltpu.make_async_copy(dst_hbm.at[pl.ds(base, EC)], dst_v.at[b],
                                  semL[b]).wait()
            pltpu.make_async_copy(e_hbm.at[pl.ds(cid * (EC // 8), EC // 8)],
                                  erow_v.at[b], semL[b]).wait()
            pltpu.async_copy(h_hbm.at[src_v.at[b]], hrow_v.at[b],
                             semG[b]).wait()
            for g in range(EC // 16):
                sl = pl.ds(g * 16, 16)
                dstS_v[b, sl] = dst_v[b, sl]

            def row(i2, rc):
                for a in range(8):
                    r = i2 * 8 + a
                    for g in range(8):
                        sl = pl.ds(g * 16, 16)
                        esl = pl.ds(a * 128 + g * 16, 16)
                        hrow_v[b, r, sl] = jnp.maximum(
                            hrow_v[b, r, sl] + erow_v[b, i2, esl], 0.0)
                return rc

            lax.fori_loop(0, EC // 8, row, 0)
            pltpu.async_copy(hrow_v.at[b], agg_sh.at[dstS_v.at[b]], semS[b],
                             add=True)

            @pl.when(cid + 2 * NW < N_CHUNKS)
            def _():
                fire_loads(b, cid + 2 * NW)

    def round_pair(j2, carry):
        for b in range(2):
            round_for(b, (2 * j2 + b) * NW + wid)
        return carry

    lax.fori_loop(0, (CHUNK_ITERS + 1) // 2, round_pair, 0)
    for b in range(2):
        pltpu.make_async_copy(hrow_v.at[b], agg_sh.at[dstS_v.at[b]],
                              semS[b]).wait()
    plsc.subcore_barrier()

    def wout(j, carry):
        r0 = s * TILE_ROWS + j * ZROWS
        pltpu.sync_copy(agg_sh.at[pl.ds(r0, ZROWS)],
                        out_hbm.at[c, pl.ds(r0, ZROWS)])
        return carry

    lax.fori_loop(0, n_copies, wout, 0)


_mp_call = pl.kernel(
    _mp_body,
    out_type=jax.ShapeDtypeStruct((NC, N, D), jnp.float32),
    mesh=plsc.VectorSubcoreMesh(core_axis_name="c", subcore_axis_name="s"),
    scratch_types=[
        pltpu.VMEM((2, EC), jnp.int32),
        pltpu.VMEM((2, EC), jnp.int32),
        pltpu.VMEM((2, EC), jnp.int32),
        pltpu.VMEM((2, EC, D), jnp.float32),
        pltpu.VMEM((2, EC // 8, 8 * D), jnp.float32),
        pltpu.VMEM_SHARED((N, D), jnp.float32),
        (pltpu.SemaphoreType.DMA, pltpu.SemaphoreType.DMA),
        (pltpu.SemaphoreType.DMA, pltpu.SemaphoreType.DMA),
        (pltpu.SemaphoreType.DMA, pltpu.SemaphoreType.DMA),
    ],
)

N_GPAD = 10240
G_CHUNKS = N_GPAD // EC
G_ITERS = -(-G_CHUNKS // NW)


def _gather_body(table_hbm, idx_hbm, out_hbm, idx_v, rows_v, semG):
    c = lax.axis_index("c")
    s = lax.axis_index("s")
    wid = s * NC + c

    def chunk(j, carry):
        cid = j * NW + wid

        @pl.when(cid < G_CHUNKS)
        def _():
            base = cid * EC
            pltpu.sync_copy(idx_hbm.at[pl.ds(base, EC)], idx_v)
            pltpu.async_copy(table_hbm.at[idx_v], rows_v, semG).wait()
            pltpu.sync_copy(rows_v, out_hbm.at[pl.ds(base, EC)])

        return carry

    lax.fori_loop(0, G_ITERS, chunk, 0)


_gather128 = pl.kernel(
    _gather_body,
    out_type=jax.ShapeDtypeStruct((N_GPAD, D), jnp.float32),
    mesh=plsc.VectorSubcoreMesh(core_axis_name="c", subcore_axis_name="s"),
    scratch_types=[
        pltpu.VMEM((EC,), jnp.int32),
        pltpu.VMEM((EC, D), jnp.float32),
        pltpu.SemaphoreType.DMA,
    ],
)


def _mm_kernel(x_ref, w_ref, b_ref, o_ref, *, relu):
    acc = jnp.dot(x_ref[...], w_ref[...], preferred_element_type=jnp.float32)
    acc = acc + b_ref[...]
    if relu:
        acc = jnp.maximum(acc, 0.0)
    o_ref[...] = acc


def _matmul(x, w, b, relu=True, block_rows=400):
    r, k = x.shape
    n = w.shape[1]
    assert r % block_rows == 0, (r, block_rows)
    out = pl.pallas_call(
        functools.partial(_mm_kernel, relu=relu),
        grid=(r // block_rows,),
        in_specs=[
            pl.BlockSpec((block_rows, k), lambda i: (i, 0)),
            pl.BlockSpec((k, n), lambda i: (0, 0)),
            pl.BlockSpec((n,), lambda i: (0,)),
        ],
        out_specs=pl.BlockSpec((block_rows, n), lambda i: (i, 0)),
        out_shape=jax.ShapeDtypeStruct((r, n), jnp.float32),
    )(x, w, b)
    return out


def _mm3_kernel(x_ref, a0_ref, a1_ref, w_ref, b_ref, o_ref):
    acc = x_ref[...] + a0_ref[...] + a1_ref[...]
    acc = jnp.dot(acc, w_ref[...], preferred_element_type=jnp.float32)
    o_ref[...] = jnp.maximum(acc + b_ref[...], 0.0)


def _mm3(x, a0, a1, w, b, block_rows=400):
    r, k = x.shape
    n = w.shape[1]
    assert r % block_rows == 0
    return pl.pallas_call(
        _mm3_kernel,
        grid=(r // block_rows,),
        in_specs=[
            pl.BlockSpec((block_rows, k), lambda i: (i, 0)),
            pl.BlockSpec((block_rows, k), lambda i: (i, 0)),
            pl.BlockSpec((block_rows, k), lambda i: (i, 0)),
            pl.BlockSpec((k, n), lambda i: (0, 0)),
            pl.BlockSpec((n,), lambda i: (0,)),
        ],
        out_specs=pl.BlockSpec((block_rows, n), lambda i: (i, 0)),
        out_shape=jax.ShapeDtypeStruct((r, n), jnp.float32),
    )(x, a0, a1, w, b)


def _seg_mean(data, seg, num):
    s = jax.ops.segment_sum(data, seg, num_segments=num)
    c = jax.ops.segment_sum(jnp.ones((data.shape[0], 1), data.dtype), seg,
                            num_segments=num)
    return s / jnp.maximum(c, 1.0)


def _mlp(h, Ws, bs, final_act):
    n = Ws.shape[0]
    for i in range(n):
        h = h @ Ws[i] + bs[i]
        if i < n - 1 or final_act:
            h = jax.nn.relu(h)
    return h


def kernel(x, edge_attr, rw_pos_enc, W_in, b_in, W_edge, b_edge, gnn_Ws,
           gnn_bs, U_W, U_b, prw_Ws, prw_bs, enc_Ws, enc_bs, pred00_Ws,
           pred00_bs, pred01_Ws, pred01_bs, pred12_Ws, pred12_bs, edge_index,
           subgraphs_nodes_mapper, subgraphs_edges_mapper, subgraphs_batch,
           fine_to_medium, medium_to_coarse, context_subgraph_idx,
           target_subgraph_idxs, target_subgraph_idxs_L1,
           target_subgraph_idxs_L2, mask):
    src, dst = edge_index[0], edge_index[1]
    map_pad = jnp.concatenate(
        [subgraphs_nodes_mapper, jnp.zeros((N_GPAD - N,), jnp.int32)])

    h = _gather128(_matmul(x, W_in, b_in), map_pad)[:N]
    W_blk = jnp.kron(jnp.eye(8, dtype=jnp.float32), W_edge)
    b_blk = jnp.tile(b_edge, 8)
    ea = edge_attr[subgraphs_edges_mapper].reshape(E // 8, 8 * DE)
    e = _matmul(ea, W_blk, b_blk, block_rows=800)

    pes = rw_pos_enc[subgraphs_nodes_mapper]
    raw_patch_pes = jax.ops.segment_max(pes, subgraphs_batch, num_segments=P0)

    agg = _mp_call(h, e, src, dst)
    h = _mm3(h, agg[0], agg[1], gnn_Ws[0], gnn_bs[0])

    batch_pad = jnp.concatenate(
        [subgraphs_batch, jnp.zeros((N_GPAD - N,), jnp.int32)])
    t = _matmul(_seg_mean(h, subgraphs_batch, P0), U_W, U_b, block_rows=P0)
    h = h + _gather128(t, batch_pad)[:N]
    node_mean = _seg_mean(h, subgraphs_nodes_mapper, N)
    h = _gather128(node_mean, map_pad)[:N]

    agg = _mp_call(h, e, src, dst)
    h = _mm3(h, agg[0], agg[1], gnn_Ws[1], gnn_bs[1])

    sx0 = _seg_mean(h, subgraphs_batch, P0)
    sx1 = _seg_mean(sx0, fine_to_medium, P1)
    pes1 = _seg_mean(raw_patch_pes, fine_to_medium, P1)
    sx2 = _seg_mean(sx1, medium_to_coarse, P2)
    pes2 = _seg_mean(pes1, medium_to_coarse, P2)
    bi0 = jnp.arange(B, dtype=jnp.int32) * 32
    bi1 = jnp.arange(B, dtype=jnp.int32) * 8
    bi2 = jnp.arange(B, dtype=jnp.int32) * 2
    ctx_idx = context_subgraph_idx + bi0
    tgt0 = target_subgraph_idxs + bi0[:, None]
    ctx_patch = sx0[ctx_idx] + jax.nn.relu(raw_patch_pes[ctx_idx] @ prw_Ws[0]
                                           + prw_bs[0])
    pe0 = jax.nn.relu(raw_patch_pes[tgt0.flatten()] @ prw_Ws[0]
                      + prw_bs[0]).reshape(B, NT0, D)
    cmask = mask[ctx_idx].astype(jnp.float32)[:, None, None]
    ctx_x0 = jax.nn.relu(ctx_patch[:, None, :] @ enc_Ws[0] + enc_bs[0]) * cmask
    tgt_x0 = sx0[tgt0.flatten()].reshape(B, NT0, D)
    tgt_x0 = jax.nn.relu(tgt_x0 @ enc_Ws[1] + enc_bs[1])
    pred0 = _mlp(ctx_x0 + pe0, pred00_Ws, pred00_bs, False)
    tgt1 = target_subgraph_idxs_L1 + bi1[:, None]
    pe1 = jax.nn.relu(pes1[tgt1.flatten()] @ prw_Ws[1]
                      + prw_bs[1]).reshape(B, NT1, D)
    tgt_x1 = sx1[tgt1.flatten()].reshape(B, NT1, D)
    tgt_x1 = jax.nn.relu(tgt_x1 @ enc_Ws[3] + enc_bs[3])
    ctx_x1 = jax.nn.relu(ctx_patch[:, None, :] @ enc_Ws[2] + enc_bs[2])
    pred1 = _mlp(ctx_x1 + pe1, pred01_Ws, pred01_bs, False)
    ctx_idx_L1 = fine_to_medium[ctx_idx]
    ctx_patch1 = sx1[ctx_idx_L1] + jax.nn.relu(pes1[ctx_idx_L1] @ prw_Ws[1]
                                               + prw_bs[1])
    tgt2 = target_subgraph_idxs_L2 + bi2[:, None]
    pe2 = jax.nn.relu(pes2[tgt2.flatten()] @ prw_Ws[2]
                      + prw_bs[2]).reshape(B, NT2, D)
    tgt_x2 = sx2[tgt2.flatten()].reshape(B, NT2, D)
    tgt_x2 = jax.nn.relu(tgt_x2 @ enc_Ws[5] + enc_bs[5])
    ctx_x2 = jax.nn.relu(ctx_patch1[:, None, :] @ enc_Ws[4] + enc_bs[4])
    pred2 = _mlp(ctx_x2 + pe2, pred12_Ws, pred12_bs, False)

    def mse(a, b):
        return jnp.mean((a - b) ** 2)

    def var_reg(p):
        std = jnp.sqrt(jnp.var(p.reshape(-1, D), axis=0) + 1e-4)
        return jnp.mean(jax.nn.relu(1.0 - std))

    loss = (1.0 * mse(pred0, tgt_x0) + 0.5 * mse(pred1, tgt_x1)
            + 0.25 * mse(pred2, tgt_x2))
    loss = loss + 0.01 * (var_reg(pred0) + var_reg(pred1) + var_reg(pred2))
    return loss


# speedup vs baseline: 1.3649x; 1.3649x over previous
import functools

import jax
import jax.numpy as jnp
from jax import lax
from jax.experimental import pallas as pl
from jax.experimental.pallas import tpu as pltpu
from jax.experimental.pallas import tpu_sc as plsc

N = 10000
E = 320000
D = 128
DE = 16
PRW = 16
B = 8
P0 = 256
P1 = 64
P2 = 16
NT0 = 4
NT1 = 4
NT2 = 1

NC = 2
NS = 16
NW = NC * NS
EC = 64
N_CHUNKS = E // EC
CHUNK_ITERS = -(-N_CHUNKS // NW)
ZROWS = 40
TILE_ROWS = 640


def _mp_body(h_hbm, e_hbm, src_hbm, dst_hbm, out_hbm,
             src_v, dst_v, dstS_v, hrow_v, erow_v, agg_sh,
             semL, semG, semS):
    c = lax.axis_index("c")
    s = lax.axis_index("s")
    wid = s * NC + c
    n_copies = jnp.where(s < NS - 1, TILE_ROWS // ZROWS, 10)

    def zrow(i, carry):
        for g in range(8):
            hrow_v[0, i, pl.ds(g * 16, 16)] = jnp.zeros((16,), jnp.float32)
        return carry

    lax.fori_loop(0, ZROWS, zrow, 0)

    def zcp(j, carry):
        pltpu.sync_copy(hrow_v.at[0, pl.ds(0, ZROWS)],
                        agg_sh.at[pl.ds(s * TILE_ROWS + j * ZROWS, ZROWS)])
        return carry

    lax.fori_loop(0, n_copies, zcp, 0)
    plsc.subcore_barrier()

    def fire_loads(b, cid):
        base = cid * EC
        pltpu.async_copy(src_hbm.at[pl.ds(base, EC)], src_v.at[b], semL[b])
        pltpu.async_copy(dst_hbm.at[pl.ds(base, EC)], dst_v.at[b], semL[b])
        pltpu.async_copy(e_hbm.at[pl.ds(base, EC)], erow_v.at[b], semL[b])

    for b in range(2):
        fire_loads(b, b * NW + wid)

    def round_for(b, cid):
        @pl.when(cid < N_CHUNKS)
        def _():
            @pl.when(cid >= 2 * NW)
            def _():
                pltpu.make_async_copy(hrow_v.at[b],
                                      agg_sh.at[dstS_v.at[b]], semS[b]).wait()

            base = cid * EC
            pltpu.make_async_copy(src_hbm.at[pl.ds(base, EC)], src_v.at[b],
                                  semL[b]).wait()
            pltpu.make_async_copy(dst_hbm.at[pl.ds(base, EC)], dst_v.at[b],
                                  semL[b]).wait()
            pltpu.make_async_copy(e_hbm.at[pl.ds(base, EC)], erow_v.at[b],
                                  semL[b]).wait()
            pltpu.async_copy(h_hbm.at[src_v.at[b]], hrow_v.at[b],
                             semG[b]).wait()
            for g in range(EC // 16):
                sl = pl.ds(g * 16, 16)
                dstS_v[b, sl] = dst_v[b, sl]

            def row(i, rc):
                for g in range(8):
                    sl = pl.ds(g * 16, 16)
                    hrow_v[b, i, sl] = jnp.maximum(
                        hrow_v[b, i, sl] + erow_v[b, i, sl], 0.0)
                return rc

            lax.fori_loop(0, EC, row, 0)
            pltpu.async_copy(hrow_v.at[b], agg_sh.at[dstS_v.at[b]], semS[b],
                             add=True)

            @pl.when(cid + 2 * NW < N_CHUNKS)
            def _():
                fire_loads(b, cid + 2 * NW)

    def round_pair(j2, carry):
        for b in range(2):
            round_for(b, (2 * j2 + b) * NW + wid)
        return carry

    lax.fori_loop(0, (CHUNK_ITERS + 1) // 2, round_pair, 0)
    for b in range(2):
        pltpu.make_async_copy(hrow_v.at[b], agg_sh.at[dstS_v.at[b]],
                              semS[b]).wait()
    plsc.subcore_barrier()

    def wout(j, carry):
        r0 = s * TILE_ROWS + j * ZROWS
        pltpu.sync_copy(agg_sh.at[pl.ds(r0, ZROWS)],
                        out_hbm.at[c, pl.ds(r0, ZROWS)])
        return carry

    lax.fori_loop(0, n_copies, wout, 0)


_mp_call = pl.kernel(
    _mp_body,
    out_type=jax.ShapeDtypeStruct((NC, N, D), jnp.float32),
    mesh=plsc.VectorSubcoreMesh(core_axis_name="c", subcore_axis_name="s"),
    scratch_types=[
        pltpu.VMEM((2, EC), jnp.int32),
        pltpu.VMEM((2, EC), jnp.int32),
        pltpu.VMEM((2, EC), jnp.int32),
        pltpu.VMEM((2, EC, D), jnp.float32),
        pltpu.VMEM((2, EC, D), jnp.float32),
        pltpu.VMEM_SHARED((N, D), jnp.float32),
        (pltpu.SemaphoreType.DMA, pltpu.SemaphoreType.DMA),
        (pltpu.SemaphoreType.DMA, pltpu.SemaphoreType.DMA),
        (pltpu.SemaphoreType.DMA, pltpu.SemaphoreType.DMA),
    ],
)

N_GPAD = 10240
G_CHUNKS = N_GPAD // EC
G_ITERS = -(-G_CHUNKS // NW)


def _gather_body(table_hbm, idx_hbm, out_hbm, idx_v, rows_v, semG):
    c = lax.axis_index("c")
    s = lax.axis_index("s")
    wid = s * NC + c

    def chunk(j, carry):
        cid = j * NW + wid

        @pl.when(cid < G_CHUNKS)
        def _():
            base = cid * EC
            pltpu.sync_copy(idx_hbm.at[pl.ds(base, EC)], idx_v)
            pltpu.async_copy(table_hbm.at[idx_v], rows_v, semG).wait()
            pltpu.sync_copy(rows_v, out_hbm.at[pl.ds(base, EC)])

        return carry

    lax.fori_loop(0, G_ITERS, chunk, 0)


_gather128 = pl.kernel(
    _gather_body,
    out_type=jax.ShapeDtypeStruct((N_GPAD, D), jnp.float32),
    mesh=plsc.VectorSubcoreMesh(core_axis_name="c", subcore_axis_name="s"),
    scratch_types=[
        pltpu.VMEM((EC,), jnp.int32),
        pltpu.VMEM((EC, D), jnp.float32),
        pltpu.SemaphoreType.DMA,
    ],
)


def _me_kernel(x_ref, w_ref, b_ref, o_ref):
    acc = jnp.dot(x_ref[...], w_ref[...], preferred_element_type=jnp.float32)
    acc = jnp.maximum(acc + b_ref[...], 0.0)
    o_ref[...] = acc.reshape(o_ref.shape)


def _matmul_e(x, w, b, block_rows=800):
    r, k = x.shape
    n = w.shape[1]
    assert r % block_rows == 0
    return pl.pallas_call(
        _me_kernel,
        grid=(r // block_rows,),
        in_specs=[
            pl.BlockSpec((block_rows, k), lambda i: (i, 0)),
            pl.BlockSpec((k, n), lambda i: (0, 0)),
            pl.BlockSpec((n,), lambda i: (0,)),
        ],
        out_specs=pl.BlockSpec((block_rows * 8, n // 8), lambda i: (i, 0)),
        out_shape=jax.ShapeDtypeStruct((r * 8, n // 8), jnp.float32),
    )(x, w, b)


def _mm_kernel(x_ref, w_ref, b_ref, o_ref, *, relu):
    acc = jnp.dot(x_ref[...], w_ref[...], preferred_element_type=jnp.float32)
    acc = acc + b_ref[...]
    if relu:
        acc = jnp.maximum(acc, 0.0)
    o_ref[...] = acc


def _matmul(x, w, b, relu=True, block_rows=400):
    r, k = x.shape
    n = w.shape[1]
    assert r % block_rows == 0, (r, block_rows)
    out = pl.pallas_call(
        functools.partial(_mm_kernel, relu=relu),
        grid=(r // block_rows,),
        in_specs=[
            pl.BlockSpec((block_rows, k), lambda i: (i, 0)),
            pl.BlockSpec((k, n), lambda i: (0, 0)),
            pl.BlockSpec((n,), lambda i: (0,)),
        ],
        out_specs=pl.BlockSpec((block_rows, n), lambda i: (i, 0)),
        out_shape=jax.ShapeDtypeStruct((r, n), jnp.float32),
    )(x, w, b)
    return out


def _mm3_kernel(x_ref, a0_ref, a1_ref, w_ref, b_ref, o_ref):
    acc = x_ref[...] + a0_ref[...] + a1_ref[...]
    acc = jnp.dot(acc, w_ref[...], preferred_element_type=jnp.float32)
    o_ref[...] = jnp.maximum(acc + b_ref[...], 0.0)


def _mm3(x, a0, a1, w, b, block_rows=400):
    r, k = x.shape
    n = w.shape[1]
    assert r % block_rows == 0
    return pl.pallas_call(
        _mm3_kernel,
        grid=(r // block_rows,),
        in_specs=[
            pl.BlockSpec((block_rows, k), lambda i: (i, 0)),
            pl.BlockSpec((block_rows, k), lambda i: (i, 0)),
            pl.BlockSpec((block_rows, k), lambda i: (i, 0)),
            pl.BlockSpec((k, n), lambda i: (0, 0)),
            pl.BlockSpec((n,), lambda i: (0,)),
        ],
        out_specs=pl.BlockSpec((block_rows, n), lambda i: (i, 0)),
        out_shape=jax.ShapeDtypeStruct((r, n), jnp.float32),
    )(x, a0, a1, w, b)


def _seg_mean(data, seg, num):
    s = jax.ops.segment_sum(data, seg, num_segments=num)
    c = jax.ops.segment_sum(jnp.ones((data.shape[0], 1), data.dtype), seg,
                            num_segments=num)
    return s / jnp.maximum(c, 1.0)


def _mlp(h, Ws, bs, final_act):
    n = Ws.shape[0]
    for i in range(n):
        h = h @ Ws[i] + bs[i]
        if i < n - 1 or final_act:
            h = jax.nn.relu(h)
    return h


def kernel(x, edge_attr, rw_pos_enc, W_in, b_in, W_edge, b_edge, gnn_Ws,
           gnn_bs, U_W, U_b, prw_Ws, prw_bs, enc_Ws, enc_bs, pred00_Ws,
           pred00_bs, pred01_Ws, pred01_bs, pred12_Ws, pred12_bs, edge_index,
           subgraphs_nodes_mapper, subgraphs_edges_mapper, subgraphs_batch,
           fine_to_medium, medium_to_coarse, context_subgraph_idx,
           target_subgraph_idxs, target_subgraph_idxs_L1,
           target_subgraph_idxs_L2, mask):
    src, dst = edge_index[0], edge_index[1]
    map_pad = jnp.concatenate(
        [subgraphs_nodes_mapper, jnp.zeros((N_GPAD - N,), jnp.int32)])

    h = _gather128(_matmul(x, W_in, b_in), map_pad)[:N]
    W_blk = jnp.kron(jnp.eye(8, dtype=jnp.float32), W_edge)
    b_blk = jnp.tile(b_edge, 8)
    ea = edge_attr[subgraphs_edges_mapper].reshape(E // 8, 8 * DE)
    e = _matmul_e(ea, W_blk, b_blk, block_rows=800)

    pes = rw_pos_enc[subgraphs_nodes_mapper]
    raw_patch_pes = jax.ops.segment_max(pes, subgraphs_batch, num_segments=P0)

    agg = _mp_call(h, e, src, dst)
    h = _mm3(h, agg[0], agg[1], gnn_Ws[0], gnn_bs[0])

    batch_pad = jnp.concatenate(
        [subgraphs_batch, jnp.zeros((N_GPAD - N,), jnp.int32)])
    t = _matmul(_seg_mean(h, subgraphs_batch, P0), U_W, U_b, block_rows=P0)
    h = h + _gather128(t, batch_pad)[:N]
    node_mean = _seg_mean(h, subgraphs_nodes_mapper, N)
    h = _gather128(node_mean, map_pad)[:N]

    agg = _mp_call(h, e, src, dst)
    h = _mm3(h, agg[0], agg[1], gnn_Ws[1], gnn_bs[1])

    sx0 = _seg_mean(h, subgraphs_batch, P0)
    sx1 = _seg_mean(sx0, fine_to_medium, P1)
    pes1 = _seg_mean(raw_patch_pes, fine_to_medium, P1)
    sx2 = _seg_mean(sx1, medium_to_coarse, P2)
    pes2 = _seg_mean(pes1, medium_to_coarse, P2)
    bi0 = jnp.arange(B, dtype=jnp.int32) * 32
    bi1 = jnp.arange(B, dtype=jnp.int32) * 8
    bi2 = jnp.arange(B, dtype=jnp.int32) * 2
    ctx_idx = context_subgraph_idx + bi0
    tgt0 = target_subgraph_idxs + bi0[:, None]
    ctx_patch = sx0[ctx_idx] + jax.nn.relu(raw_patch_pes[ctx_idx] @ prw_Ws[0]
                                           + prw_bs[0])
    pe0 = jax.nn.relu(raw_patch_pes[tgt0.flatten()] @ prw_Ws[0]
                      + prw_bs[0]).reshape(B, NT0, D)
    cmask = mask[ctx_idx].astype(jnp.float32)[:, None, None]
    ctx_x0 = jax.nn.relu(ctx_patch[:, None, :] @ enc_Ws[0] + enc_bs[0]) * cmask
    tgt_x0 = sx0[tgt0.flatten()].reshape(B, NT0, D)
    tgt_x0 = jax.nn.relu(tgt_x0 @ enc_Ws[1] + enc_bs[1])
    pred0 = _mlp(ctx_x0 + pe0, pred00_Ws, pred00_bs, False)
    tgt1 = target_subgraph_idxs_L1 + bi1[:, None]
    pe1 = jax.nn.relu(pes1[tgt1.flatten()] @ prw_Ws[1]
                      + prw_bs[1]).reshape(B, NT1, D)
    tgt_x1 = sx1[tgt1.flatten()].reshape(B, NT1, D)
    tgt_x1 = jax.nn.relu(tgt_x1 @ enc_Ws[3] + enc_bs[3])
    ctx_x1 = jax.nn.relu(ctx_patch[:, None, :] @ enc_Ws[2] + enc_bs[2])
    pred1 = _mlp(ctx_x1 + pe1, pred01_Ws, pred01_bs, False)
    ctx_idx_L1 = fine_to_medium[ctx_idx]
    ctx_patch1 = sx1[ctx_idx_L1] + jax.nn.relu(pes1[ctx_idx_L1] @ prw_Ws[1]
                                               + prw_bs[1])
    tgt2 = target_subgraph_idxs_L2 + bi2[:, None]
    pe2 = jax.nn.relu(pes2[tgt2.flatten()] @ prw_Ws[2]
                      + prw_bs[2]).reshape(B, NT2, D)
    tgt_x2 = sx2[tgt2.flatten()].reshape(B, NT2, D)
    tgt_x2 = jax.nn.relu(tgt_x2 @ enc_Ws[5] + enc_bs[5])
    ctx_x2 = jax.nn.relu(ctx_patch1[:, None, :] @ enc_Ws[4] + enc_bs[4])
    pred2 = _mlp(ctx_x2 + pe2, pred12_Ws, pred12_bs, False)

    def mse(a, b):
        return jnp.mean((a - b) ** 2)

    def var_reg(p):
        std = jnp.sqrt(jnp.var(p.reshape(-1, D), axis=0) + 1e-4)
        return jnp.mean(jax.nn.relu(1.0 - std))

    loss = (1.0 * mse(pred0, tgt_x0) + 0.5 * mse(pred1, tgt_x1)
            + 0.25 * mse(pred2, tgt_x2))
    loss = loss + 0.01 * (var_reg(pred0) + var_reg(pred1) + var_reg(pred2))
    return loss


# speedup vs baseline: 1.5759x; 1.1546x over previous
import functools

import jax
import jax.numpy as jnp
from jax import lax
from jax.experimental import pallas as pl
from jax.experimental.pallas import tpu as pltpu
from jax.experimental.pallas import tpu_sc as plsc

N = 10000
E = 320000
D = 128
DE = 16
PRW = 16
B = 8
P0 = 256
P1 = 64
P2 = 16
NT0 = 4
NT1 = 4
NT2 = 1

NC = 2
NS = 16
NW = NC * NS
EC = 64
N_CHUNKS = E // EC
CHUNK_ITERS = -(-N_CHUNKS // NW)
ZROWS = 40
TILE_ROWS = 640


def _mp_body(h_hbm, e_hbm, src_hbm, dst_hbm, emap_hbm, out_hbm,
             src_v, dst_v, emap_v, dstS_v, hrow_v, erow_v, agg_sh,
             semL, semG, semE, semS):
    c = lax.axis_index("c")
    s = lax.axis_index("s")
    wid = s * NC + c
    n_copies = jnp.where(s < NS - 1, TILE_ROWS // ZROWS, 10)

    def zrow(i, carry):
        for g in range(8):
            hrow_v[0, i, pl.ds(g * 16, 16)] = jnp.zeros((16,), jnp.float32)
        return carry

    lax.fori_loop(0, ZROWS, zrow, 0)

    def zcp(j, carry):
        pltpu.sync_copy(hrow_v.at[0, pl.ds(0, ZROWS)],
                        agg_sh.at[pl.ds(s * TILE_ROWS + j * ZROWS, ZROWS)])
        return carry

    lax.fori_loop(0, n_copies, zcp, 0)
    plsc.subcore_barrier()

    def fire_loads(b, cid):
        base = cid * EC
        pltpu.async_copy(src_hbm.at[pl.ds(base, EC)], src_v.at[b], semL[b])
        pltpu.async_copy(dst_hbm.at[pl.ds(base, EC)], dst_v.at[b], semL[b])
        pltpu.async_copy(emap_hbm.at[pl.ds(base, EC)], emap_v.at[b], semL[b])

    for b in range(2):
        fire_loads(b, b * NW + wid)

    def round_for(b, cid):
        @pl.when(cid < N_CHUNKS)
        def _():
            @pl.when(cid >= 2 * NW)
            def _():
                pltpu.make_async_copy(hrow_v.at[b],
                                      agg_sh.at[dstS_v.at[b]], semS[b]).wait()

            base = cid * EC
            pltpu.make_async_copy(src_hbm.at[pl.ds(base, EC)], src_v.at[b],
                                  semL[b]).wait()
            pltpu.make_async_copy(dst_hbm.at[pl.ds(base, EC)], dst_v.at[b],
                                  semL[b]).wait()
            pltpu.make_async_copy(emap_hbm.at[pl.ds(base, EC)], emap_v.at[b],
                                  semL[b]).wait()
            gh = pltpu.async_copy(h_hbm.at[src_v.at[b]], hrow_v.at[b],
                                  semG[b])
            ge = pltpu.async_copy(e_hbm.at[emap_v.at[b]], erow_v.at[b],
                                  semE[b])
            gh.wait()
            ge.wait()
            for g in range(EC // 16):
                sl = pl.ds(g * 16, 16)
                dstS_v[b, sl] = dst_v[b, sl]

            def row(i, rc):
                for g in range(8):
                    sl = pl.ds(g * 16, 16)
                    hrow_v[b, i, sl] = jnp.maximum(
                        hrow_v[b, i, sl] + erow_v[b, i, sl], 0.0)
                return rc

            lax.fori_loop(0, EC, row, 0)
            pltpu.async_copy(hrow_v.at[b], agg_sh.at[dstS_v.at[b]], semS[b],
                             add=True)

            @pl.when(cid + 2 * NW < N_CHUNKS)
            def _():
                fire_loads(b, cid + 2 * NW)

    def round_pair(j2, carry):
        for b in range(2):
            round_for(b, (2 * j2 + b) * NW + wid)
        return carry

    lax.fori_loop(0, (CHUNK_ITERS + 1) // 2, round_pair, 0)
    for b in range(2):
        pltpu.make_async_copy(hrow_v.at[b], agg_sh.at[dstS_v.at[b]],
                              semS[b]).wait()
    plsc.subcore_barrier()

    def wout(j, carry):
        r0 = s * TILE_ROWS + j * ZROWS
        pltpu.sync_copy(agg_sh.at[pl.ds(r0, ZROWS)],
                        out_hbm.at[c, pl.ds(r0, ZROWS)])
        return carry

    lax.fori_loop(0, n_copies, wout, 0)


_mp_call = pl.kernel(
    _mp_body,
    out_type=jax.ShapeDtypeStruct((NC, N, D), jnp.float32),
    mesh=plsc.VectorSubcoreMesh(core_axis_name="c", subcore_axis_name="s"),
    scratch_types=[
        pltpu.VMEM((2, EC), jnp.int32),
        pltpu.VMEM((2, EC), jnp.int32),
        pltpu.VMEM((2, EC), jnp.int32),
        pltpu.VMEM((2, EC), jnp.int32),
        pltpu.VMEM((2, EC, D), jnp.float32),
        pltpu.VMEM((2, EC, D), jnp.float32),
        pltpu.VMEM_SHARED((N, D), jnp.float32),
        (pltpu.SemaphoreType.DMA, pltpu.SemaphoreType.DMA),
        (pltpu.SemaphoreType.DMA, pltpu.SemaphoreType.DMA),
        (pltpu.SemaphoreType.DMA, pltpu.SemaphoreType.DMA),
        (pltpu.SemaphoreType.DMA, pltpu.SemaphoreType.DMA),
    ],
)

N_GPAD = 10240
G_CHUNKS = N_GPAD // EC
G_ITERS = -(-G_CHUNKS // NW)


def _gather_body(table_hbm, idx_hbm, out_hbm, idx_v, rows_v, semG):
    c = lax.axis_index("c")
    s = lax.axis_index("s")
    wid = s * NC + c

    def chunk(j, carry):
        cid = j * NW + wid

        @pl.when(cid < G_CHUNKS)
        def _():
            base = cid * EC
            pltpu.sync_copy(idx_hbm.at[pl.ds(base, EC)], idx_v)
            pltpu.async_copy(table_hbm.at[idx_v], rows_v, semG).wait()
            pltpu.sync_copy(rows_v, out_hbm.at[pl.ds(base, EC)])

        return carry

    lax.fori_loop(0, G_ITERS, chunk, 0)


_gather128 = pl.kernel(
    _gather_body,
    out_type=jax.ShapeDtypeStruct((N_GPAD, D), jnp.float32),
    mesh=plsc.VectorSubcoreMesh(core_axis_name="c", subcore_axis_name="s"),
    scratch_types=[
        pltpu.VMEM((EC,), jnp.int32),
        pltpu.VMEM((EC, D), jnp.float32),
        pltpu.SemaphoreType.DMA,
    ],
)


def _me_kernel(x_ref, w_ref, b_ref, o_ref):
    acc = jnp.dot(x_ref[...], w_ref[...], preferred_element_type=jnp.float32)
    acc = jnp.maximum(acc + b_ref[...], 0.0)
    o_ref[...] = acc.reshape(o_ref.shape)


def _matmul_e(x, w, b, block_rows=800):
    r, k = x.shape
    n = w.shape[1]
    assert r % block_rows == 0
    return pl.pallas_call(
        _me_kernel,
        grid=(r // block_rows,),
        in_specs=[
            pl.BlockSpec((block_rows, k), lambda i: (i, 0)),
            pl.BlockSpec((k, n), lambda i: (0, 0)),
            pl.BlockSpec((n,), lambda i: (0,)),
        ],
        out_specs=pl.BlockSpec((block_rows * 8, n // 8), lambda i: (i, 0)),
        out_shape=jax.ShapeDtypeStruct((r * 8, n // 8), jnp.float32),
    )(x, w, b)


def _mm_kernel(x_ref, w_ref, b_ref, o_ref, *, relu):
    acc = jnp.dot(x_ref[...], w_ref[...], preferred_element_type=jnp.float32)
    acc = acc + b_ref[...]
    if relu:
        acc = jnp.maximum(acc, 0.0)
    o_ref[...] = acc


def _matmul(x, w, b, relu=True, block_rows=400):
    r, k = x.shape
    n = w.shape[1]
    assert r % block_rows == 0, (r, block_rows)
    out = pl.pallas_call(
        functools.partial(_mm_kernel, relu=relu),
        grid=(r // block_rows,),
        in_specs=[
            pl.BlockSpec((block_rows, k), lambda i: (i, 0)),
            pl.BlockSpec((k, n), lambda i: (0, 0)),
            pl.BlockSpec((n,), lambda i: (0,)),
        ],
        out_specs=pl.BlockSpec((block_rows, n), lambda i: (i, 0)),
        out_shape=jax.ShapeDtypeStruct((r, n), jnp.float32),
    )(x, w, b)
    return out


def _mm3_kernel(x_ref, a0_ref, a1_ref, w_ref, b_ref, o_ref):
    acc = x_ref[...] + a0_ref[...] + a1_ref[...]
    acc = jnp.dot(acc, w_ref[...], preferred_element_type=jnp.float32)
    o_ref[...] = jnp.maximum(acc + b_ref[...], 0.0)


def _mm3(x, a0, a1, w, b, block_rows=400):
    r, k = x.shape
    n = w.shape[1]
    assert r % block_rows == 0
    return pl.pallas_call(
        _mm3_kernel,
        grid=(r // block_rows,),
        in_specs=[
            pl.BlockSpec((block_rows, k), lambda i: (i, 0)),
            pl.BlockSpec((block_rows, k), lambda i: (i, 0)),
            pl.BlockSpec((block_rows, k), lambda i: (i, 0)),
            pl.BlockSpec((k, n), lambda i: (0, 0)),
            pl.BlockSpec((n,), lambda i: (0,)),
        ],
        out_specs=pl.BlockSpec((block_rows, n), lambda i: (i, 0)),
        out_shape=jax.ShapeDtypeStruct((r, n), jnp.float32),
    )(x, a0, a1, w, b)


def _seg_mean(data, seg, num):
    s = jax.ops.segment_sum(data, seg, num_segments=num)
    c = jax.ops.segment_sum(jnp.ones((data.shape[0], 1), data.dtype), seg,
                            num_segments=num)
    return s / jnp.maximum(c, 1.0)


def _mlp(h, Ws, bs, final_act):
    n = Ws.shape[0]
    for i in range(n):
        h = h @ Ws[i] + bs[i]
        if i < n - 1 or final_act:
            h = jax.nn.relu(h)
    return h


def kernel(x, edge_attr, rw_pos_enc, W_in, b_in, W_edge, b_edge, gnn_Ws,
           gnn_bs, U_W, U_b, prw_Ws, prw_bs, enc_Ws, enc_bs, pred00_Ws,
           pred00_bs, pred01_Ws, pred01_bs, pred12_Ws, pred12_bs, edge_index,
           subgraphs_nodes_mapper, subgraphs_edges_mapper, subgraphs_batch,
           fine_to_medium, medium_to_coarse, context_subgraph_idx,
           target_subgraph_idxs, target_subgraph_idxs_L1,
           target_subgraph_idxs_L2, mask):
    src, dst = edge_index[0], edge_index[1]
    map_pad = jnp.concatenate(
        [subgraphs_nodes_mapper, jnp.zeros((N_GPAD - N,), jnp.int32)])

    h = _gather128(_matmul(x, W_in, b_in), map_pad)[:N]
    W_blk = jnp.kron(jnp.eye(8, dtype=jnp.float32), W_edge)
    b_blk = jnp.tile(b_edge, 8)
    e = _matmul_e(edge_attr.reshape(E // 8, 8 * DE), W_blk, b_blk,
                  block_rows=800)

    pes = rw_pos_enc[subgraphs_nodes_mapper]
    raw_patch_pes = jax.ops.segment_max(pes, subgraphs_batch, num_segments=P0)

    agg = _mp_call(h, e, src, dst, subgraphs_edges_mapper)
    h = _mm3(h, agg[0], agg[1], gnn_Ws[0], gnn_bs[0])

    batch_pad = jnp.concatenate(
        [subgraphs_batch, jnp.zeros((N_GPAD - N,), jnp.int32)])
    t = _matmul(_seg_mean(h, subgraphs_batch, P0), U_W, U_b, block_rows=P0)
    h = h + _gather128(t, batch_pad)[:N]
    node_mean = _seg_mean(h, subgraphs_nodes_mapper, N)
    h = _gather128(node_mean, map_pad)[:N]

    agg = _mp_call(h, e, src, dst, subgraphs_edges_mapper)
    h = _mm3(h, agg[0], agg[1], gnn_Ws[1], gnn_bs[1])

    sx0 = _seg_mean(h, subgraphs_batch, P0)
    sx1 = _seg_mean(sx0, fine_to_medium, P1)
    pes1 = _seg_mean(raw_patch_pes, fine_to_medium, P1)
    sx2 = _seg_mean(sx1, medium_to_coarse, P2)
    pes2 = _seg_mean(pes1, medium_to_coarse, P2)
    bi0 = jnp.arange(B, dtype=jnp.int32) * 32
    bi1 = jnp.arange(B, dtype=jnp.int32) * 8
    bi2 = jnp.arange(B, dtype=jnp.int32) * 2
    ctx_idx = context_subgraph_idx + bi0
    tgt0 = target_subgraph_idxs + bi0[:, None]
    ctx_patch = sx0[ctx_idx] + jax.nn.relu(raw_patch_pes[ctx_idx] @ prw_Ws[0]
                                           + prw_bs[0])
    pe0 = jax.nn.relu(raw_patch_pes[tgt0.flatten()] @ prw_Ws[0]
                      + prw_bs[0]).reshape(B, NT0, D)
    cmask = mask[ctx_idx].astype(jnp.float32)[:, None, None]
    ctx_x0 = jax.nn.relu(ctx_patch[:, None, :] @ enc_Ws[0] + enc_bs[0]) * cmask
    tgt_x0 = sx0[tgt0.flatten()].reshape(B, NT0, D)
    tgt_x0 = jax.nn.relu(tgt_x0 @ enc_Ws[1] + enc_bs[1])
    pred0 = _mlp(ctx_x0 + pe0, pred00_Ws, pred00_bs, False)
    tgt1 = target_subgraph_idxs_L1 + bi1[:, None]
    pe1 = jax.nn.relu(pes1[tgt1.flatten()] @ prw_Ws[1]
                      + prw_bs[1]).reshape(B, NT1, D)
    tgt_x1 = sx1[tgt1.flatten()].reshape(B, NT1, D)
    tgt_x1 = jax.nn.relu(tgt_x1 @ enc_Ws[3] + enc_bs[3])
    ctx_x1 = jax.nn.relu(ctx_patch[:, None, :] @ enc_Ws[2] + enc_bs[2])
    pred1 = _mlp(ctx_x1 + pe1, pred01_Ws, pred01_bs, False)
    ctx_idx_L1 = fine_to_medium[ctx_idx]
    ctx_patch1 = sx1[ctx_idx_L1] + jax.nn.relu(pes1[ctx_idx_L1] @ prw_Ws[1]
                                               + prw_bs[1])
    tgt2 = target_subgraph_idxs_L2 + bi2[:, None]
    pe2 = jax.nn.relu(pes2[tgt2.flatten()] @ prw_Ws[2]
                      + prw_bs[2]).reshape(B, NT2, D)
    tgt_x2 = sx2[tgt2.flatten()].reshape(B, NT2, D)
    tgt_x2 = jax.nn.relu(tgt_x2 @ enc_Ws[5] + enc_bs[5])
    ctx_x2 = jax.nn.relu(ctx_patch1[:, None, :] @ enc_Ws[4] + enc_bs[4])
    pred2 = _mlp(ctx_x2 + pe2, pred12_Ws, pred12_bs, False)

    def mse(a, b):
        return jnp.mean((a - b) ** 2)

    def var_reg(p):
        std = jnp.sqrt(jnp.var(p.reshape(-1, D), axis=0) + 1e-4)
        return jnp.mean(jax.nn.relu(1.0 - std))

    loss = (1.0 * mse(pred0, tgt_x0) + 0.5 * mse(pred1, tgt_x1)
            + 0.25 * mse(pred2, tgt_x2))
    loss = loss + 0.01 * (var_reg(pred0) + var_reg(pred1) + var_reg(pred2))
    return loss


# speedup vs baseline: 1.8483x; 1.1728x over previous
import functools

import jax
import jax.numpy as jnp
from jax import lax
from jax.experimental import pallas as pl
from jax.experimental.pallas import tpu as pltpu
from jax.experimental.pallas import tpu_sc as plsc

N = 10000
E = 320000
D = 128
DE = 16
PRW = 16
B = 8
P0 = 256
P1 = 64
P2 = 16
NT0 = 4
NT1 = 4
NT2 = 1

NC = 2
NS = 16
NW = NC * NS
EC = 64
N_CHUNKS = E // EC
CHUNK_ITERS = -(-N_CHUNKS // NW)
ZROWS = 40
TILE_ROWS = 640


def _mp_body(h_hbm, e_hbm, src_hbm, dst_hbm, emap_hbm, out_hbm,
             src_v, dst_v, emap_v, dstS_v, hrow_v, erow_v, agg_sh,
             semL, semG, semE, semS):
    c = lax.axis_index("c")
    s = lax.axis_index("s")
    wid = s * NC + c
    n_copies = jnp.where(s < NS - 1, TILE_ROWS // ZROWS, 10)

    def zrow(i, carry):
        for g in range(8):
            hrow_v[0, i, pl.ds(g * 16, 16)] = jnp.zeros((16,), jnp.float32)
        return carry

    lax.fori_loop(0, ZROWS, zrow, 0)

    def zcp(j, carry):
        pltpu.sync_copy(hrow_v.at[0, pl.ds(0, ZROWS)],
                        agg_sh.at[pl.ds(s * TILE_ROWS + j * ZROWS, ZROWS)])
        return carry

    lax.fori_loop(0, n_copies, zcp, 0)
    plsc.subcore_barrier()

    def fire_loads(b, cid):
        base = cid * EC
        pltpu.async_copy(src_hbm.at[pl.ds(base, EC)], src_v.at[b], semL[b])
        pltpu.async_copy(dst_hbm.at[pl.ds(base, EC)], dst_v.at[b], semL[b])
        pltpu.async_copy(emap_hbm.at[pl.ds(base, EC)], emap_v.at[b], semL[b])

    for b in range(2):
        fire_loads(b, b * NW + wid)

    def round_for(b, cid):
        @pl.when(cid < N_CHUNKS)
        def _():
            @pl.when(cid >= 2 * NW)
            def _():
                pltpu.make_async_copy(hrow_v.at[b],
                                      agg_sh.at[dstS_v.at[b]], semS[b]).wait()

            base = cid * EC
            pltpu.make_async_copy(src_hbm.at[pl.ds(base, EC)], src_v.at[b],
                                  semL[b]).wait()
            pltpu.make_async_copy(dst_hbm.at[pl.ds(base, EC)], dst_v.at[b],
                                  semL[b]).wait()
            pltpu.make_async_copy(emap_hbm.at[pl.ds(base, EC)], emap_v.at[b],
                                  semL[b]).wait()
            gh = pltpu.async_copy(h_hbm.at[src_v.at[b]], hrow_v.at[b],
                                  semG[b])
            ge = pltpu.async_copy(e_hbm.at[emap_v.at[b]], erow_v.at[b],
                                  semE[b])
            gh.wait()
            ge.wait()
            for g in range(EC // 16):
                sl = pl.ds(g * 16, 16)
                dstS_v[b, sl] = dst_v[b, sl]

            def row(i, rc):
                for g in range(8):
                    sl = pl.ds(g * 16, 16)
                    hrow_v[b, i, sl] = jnp.maximum(
                        hrow_v[b, i, sl] + erow_v[b, i, sl], 0.0)
                return rc

            lax.fori_loop(0, EC, row, 0)
            pltpu.async_copy(hrow_v.at[b], agg_sh.at[dstS_v.at[b]], semS[b],
                             add=True)

            @pl.when(cid + 2 * NW < N_CHUNKS)
            def _():
                fire_loads(b, cid + 2 * NW)

    def round_pair(j2, carry):
        for b in range(2):
            round_for(b, (2 * j2 + b) * NW + wid)
        return carry

    lax.fori_loop(0, (CHUNK_ITERS + 1) // 2, round_pair, 0)
    for b in range(2):
        pltpu.make_async_copy(hrow_v.at[b], agg_sh.at[dstS_v.at[b]],
                              semS[b]).wait()
    plsc.subcore_barrier()

    def wout(j, carry):
        r0 = s * TILE_ROWS + j * ZROWS
        pltpu.sync_copy(agg_sh.at[pl.ds(r0, ZROWS)],
                        out_hbm.at[c, pl.ds(r0, ZROWS)])
        return carry

    lax.fori_loop(0, n_copies, wout, 0)


_mp_call = pl.kernel(
    _mp_body,
    out_type=jax.ShapeDtypeStruct((NC, N, D), jnp.float32),
    mesh=plsc.VectorSubcoreMesh(core_axis_name="c", subcore_axis_name="s"),
    scratch_types=[
        pltpu.VMEM((2, EC), jnp.int32),
        pltpu.VMEM((2, EC), jnp.int32),
        pltpu.VMEM((2, EC), jnp.int32),
        pltpu.VMEM((2, EC), jnp.int32),
        pltpu.VMEM((2, EC, D), jnp.float32),
        pltpu.VMEM((2, EC, D), jnp.float32),
        pltpu.VMEM_SHARED((N, D), jnp.float32),
        (pltpu.SemaphoreType.DMA, pltpu.SemaphoreType.DMA),
        (pltpu.SemaphoreType.DMA, pltpu.SemaphoreType.DMA),
        (pltpu.SemaphoreType.DMA, pltpu.SemaphoreType.DMA),
        (pltpu.SemaphoreType.DMA, pltpu.SemaphoreType.DMA),
    ],
)

N_GPAD = 10240
G_CHUNKS = N_GPAD // EC
G_ITERS = -(-G_CHUNKS // NW)


def _gather_body(table_hbm, idx_hbm, out_hbm, idx_v, rows_v, semG):
    c = lax.axis_index("c")
    s = lax.axis_index("s")
    wid = s * NC + c

    def chunk(j, carry):
        cid = j * NW + wid

        @pl.when(cid < G_CHUNKS)
        def _():
            base = cid * EC
            pltpu.sync_copy(idx_hbm.at[pl.ds(base, EC)], idx_v)
            pltpu.async_copy(table_hbm.at[idx_v], rows_v, semG).wait()
            pltpu.sync_copy(rows_v, out_hbm.at[pl.ds(base, EC)])

        return carry

    lax.fori_loop(0, G_ITERS, chunk, 0)


_gather128 = pl.kernel(
    _gather_body,
    out_type=jax.ShapeDtypeStruct((N_GPAD, D), jnp.float32),
    mesh=plsc.VectorSubcoreMesh(core_axis_name="c", subcore_axis_name="s"),
    scratch_types=[
        pltpu.VMEM((EC,), jnp.int32),
        pltpu.VMEM((EC, D), jnp.float32),
        pltpu.SemaphoreType.DMA,
    ],
)


def _me_kernel(x_ref, w_ref, b_ref, o_ref):
    acc = jnp.dot(x_ref[...], w_ref[...], preferred_element_type=jnp.float32)
    acc = jnp.maximum(acc + b_ref[...], 0.0)
    o_ref[...] = acc.reshape(o_ref.shape)


def _matmul_e(x, w, b, block_rows=800):
    r, k = x.shape
    n = w.shape[1]
    assert r % block_rows == 0
    return pl.pallas_call(
        _me_kernel,
        grid=(r // block_rows,),
        in_specs=[
            pl.BlockSpec((block_rows, k), lambda i: (i, 0)),
            pl.BlockSpec((k, n), lambda i: (0, 0)),
            pl.BlockSpec((n,), lambda i: (0,)),
        ],
        out_specs=pl.BlockSpec((block_rows * 8, n // 8), lambda i: (i, 0)),
        out_shape=jax.ShapeDtypeStruct((r * 8, n // 8), jnp.float32),
    )(x, w, b)


RC = 80
S_CHUNKS = N // RC
S_ITERS = -(-S_CHUNKS // NW)


def _ssn_body(x_hbm, idx_hbm, sum_hbm,
              idx_v, rows_v, sum_sh, semG):
    c = lax.axis_index("c")
    s = lax.axis_index("s")
    wid = s * NC + c
    n_copies = jnp.where(s < NS - 1, TILE_ROWS // ZROWS, 10)

    def zrow(i, carry):
        for g in range(8):
            rows_v[i, pl.ds(g * 16, 16)] = jnp.zeros((16,), jnp.float32)
        return carry

    lax.fori_loop(0, ZROWS, zrow, 0)

    def zcp(j, carry):
        r0 = s * TILE_ROWS + j * ZROWS
        pltpu.sync_copy(rows_v.at[pl.ds(0, ZROWS)],
                        sum_sh.at[pl.ds(r0, ZROWS)])
        return carry

    lax.fori_loop(0, n_copies, zcp, 0)
    plsc.subcore_barrier()

    def chunk(j, carry):
        cid = j * NW + wid

        @pl.when(cid < S_CHUNKS)
        def _():
            base = cid * RC
            pltpu.sync_copy(idx_hbm.at[pl.ds(base, RC)], idx_v)
            pltpu.sync_copy(x_hbm.at[pl.ds(base, RC)], rows_v)
            pltpu.sync_copy(rows_v, sum_sh.at[idx_v], add=True)

        return carry

    lax.fori_loop(0, S_ITERS, chunk, 0)
    plsc.subcore_barrier()

    def wout(j, carry):
        r0 = s * TILE_ROWS + j * ZROWS
        pltpu.sync_copy(sum_sh.at[pl.ds(r0, ZROWS)],
                        sum_hbm.at[c, pl.ds(r0, ZROWS)])
        return carry

    lax.fori_loop(0, n_copies, wout, 0)


_ssn_call = pl.kernel(
    _ssn_body,
    out_type=jax.ShapeDtypeStruct((NC, N, D), jnp.float32),
    mesh=plsc.VectorSubcoreMesh(core_axis_name="c", subcore_axis_name="s"),
    scratch_types=[
        pltpu.VMEM((RC,), jnp.int32),
        pltpu.VMEM((RC, D), jnp.float32),
        pltpu.VMEM_SHARED((N, D), jnp.float32),
        pltpu.SemaphoreType.DMA,
    ],
)


def _ss256_kernel(idx_ref, x_ref, sum_ref, cnt_ref):
    i = pl.program_id(0)
    idx = idx_ref[0, 0, :]
    oh = (idx[:, None]
          == lax.broadcasted_iota(jnp.int32, (idx.shape[0], P0), 1)
          ).astype(jnp.float32)
    part = lax.dot_general(oh, x_ref[...], (((0,), (0,)), ((), ())),
                           preferred_element_type=jnp.float32)
    pc = jnp.sum(oh, axis=0)

    @pl.when(i == 0)
    def _():
        sum_ref[...] = part
        cnt_ref[...] = pc

    @pl.when(i > 0)
    def _():
        sum_ref[...] = sum_ref[...] + part
        cnt_ref[...] = cnt_ref[...] + pc


def _segsum256(data, idx, block_rows=2000):
    r = data.shape[0]
    assert r % block_rows == 0
    return pl.pallas_call(
        _ss256_kernel,
        grid=(r // block_rows,),
        in_specs=[
            pl.BlockSpec((1, 1, block_rows), lambda i: (i, 0, 0)),
            pl.BlockSpec((block_rows, D), lambda i: (i, 0)),
        ],
        out_specs=(pl.BlockSpec((P0, D), lambda i: (0, 0)),
                   pl.BlockSpec((P0,), lambda i: (0,))),
        out_shape=(jax.ShapeDtypeStruct((P0, D), jnp.float32),
                   jax.ShapeDtypeStruct((P0,), jnp.float32)),
    )(idx.reshape(r // block_rows, 1, block_rows), data)

def _mm_kernel(x_ref, w_ref, b_ref, o_ref, *, relu):
    acc = jnp.dot(x_ref[...], w_ref[...], preferred_element_type=jnp.float32)
    acc = acc + b_ref[...]
    if relu:
        acc = jnp.maximum(acc, 0.0)
    o_ref[...] = acc


def _matmul(x, w, b, relu=True, block_rows=400):
    r, k = x.shape
    n = w.shape[1]
    assert r % block_rows == 0, (r, block_rows)
    out = pl.pallas_call(
        functools.partial(_mm_kernel, relu=relu),
        grid=(r // block_rows,),
        in_specs=[
            pl.BlockSpec((block_rows, k), lambda i: (i, 0)),
            pl.BlockSpec((k, n), lambda i: (0, 0)),
            pl.BlockSpec((n,), lambda i: (0,)),
        ],
        out_specs=pl.BlockSpec((block_rows, n), lambda i: (i, 0)),
        out_shape=jax.ShapeDtypeStruct((r, n), jnp.float32),
    )(x, w, b)
    return out


def _mm3_kernel(x_ref, a0_ref, a1_ref, w_ref, b_ref, o_ref):
    acc = x_ref[...] + a0_ref[...] + a1_ref[...]
    acc = jnp.dot(acc, w_ref[...], preferred_element_type=jnp.float32)
    o_ref[...] = jnp.maximum(acc + b_ref[...], 0.0)


def _mm3(x, a0, a1, w, b, block_rows=400):
    r, k = x.shape
    n = w.shape[1]
    assert r % block_rows == 0
    return pl.pallas_call(
        _mm3_kernel,
        grid=(r // block_rows,),
        in_specs=[
            pl.BlockSpec((block_rows, k), lambda i: (i, 0)),
            pl.BlockSpec((block_rows, k), lambda i: (i, 0)),
            pl.BlockSpec((block_rows, k), lambda i: (i, 0)),
            pl.BlockSpec((k, n), lambda i: (0, 0)),
            pl.BlockSpec((n,), lambda i: (0,)),
        ],
        out_specs=pl.BlockSpec((block_rows, n), lambda i: (i, 0)),
        out_shape=jax.ShapeDtypeStruct((r, n), jnp.float32),
    )(x, a0, a1, w, b)


def _seg_mean(data, seg, num):
    s = jax.ops.segment_sum(data, seg, num_segments=num)
    c = jax.ops.segment_sum(jnp.ones((data.shape[0], 1), data.dtype), seg,
                            num_segments=num)
    return s / jnp.maximum(c, 1.0)


def _mlp(h, Ws, bs, final_act):
    n = Ws.shape[0]
    for i in range(n):
        h = h @ Ws[i] + bs[i]
        if i < n - 1 or final_act:
            h = jax.nn.relu(h)
    return h


def kernel(x, edge_attr, rw_pos_enc, W_in, b_in, W_edge, b_edge, gnn_Ws,
           gnn_bs, U_W, U_b, prw_Ws, prw_bs, enc_Ws, enc_bs, pred00_Ws,
           pred00_bs, pred01_Ws, pred01_bs, pred12_Ws, pred12_bs, edge_index,
           subgraphs_nodes_mapper, subgraphs_edges_mapper, subgraphs_batch,
           fine_to_medium, medium_to_coarse, context_subgraph_idx,
           target_subgraph_idxs, target_subgraph_idxs_L1,
           target_subgraph_idxs_L2, mask):
    src, dst = edge_index[0], edge_index[1]
    map_pad = jnp.concatenate(
        [subgraphs_nodes_mapper, jnp.zeros((N_GPAD - N,), jnp.int32)])

    h = _gather128(_matmul(x, W_in, b_in), map_pad)[:N]
    W_blk = jnp.kron(jnp.eye(8, dtype=jnp.float32), W_edge)
    b_blk = jnp.tile(b_edge, 8)
    e = _matmul_e(edge_attr.reshape(E // 8, 8 * DE), W_blk, b_blk,
                  block_rows=800)

    pes = rw_pos_enc[subgraphs_nodes_mapper]
    raw_patch_pes = jax.ops.segment_max(pes, subgraphs_batch, num_segments=P0)

    agg = _mp_call(h, e, src, dst, subgraphs_edges_mapper)
    h = _mm3(h, agg[0], agg[1], gnn_Ws[0], gnn_bs[0])

    batch_pad = jnp.concatenate(
        [subgraphs_batch, jnp.zeros((N_GPAD - N,), jnp.int32)])
    bsum, bcnt = _segsum256(h, subgraphs_batch)
    t = _matmul(bsum / jnp.maximum(bcnt, 1.0)[:, None], U_W, U_b,
                block_rows=P0)
    h = h + _gather128(t, batch_pad)[:N]
    nsum = _ssn_call(h, subgraphs_nodes_mapper)
    ncnt = jax.ops.segment_sum(jnp.ones((N, 1), jnp.float32),
                               subgraphs_nodes_mapper, num_segments=N)
    node_mean = (nsum[0] + nsum[1]) / jnp.maximum(ncnt, 1.0)
    h = _gather128(node_mean, map_pad)[:N]

    agg = _mp_call(h, e, src, dst, subgraphs_edges_mapper)
    h = _mm3(h, agg[0], agg[1], gnn_Ws[1], gnn_bs[1])

    s0sum, s0cnt = _segsum256(h, subgraphs_batch)
    sx0 = s0sum / jnp.maximum(s0cnt, 1.0)[:, None]
    sx1 = _seg_mean(sx0, fine_to_medium, P1)
    pes1 = _seg_mean(raw_patch_pes, fine_to_medium, P1)
    sx2 = _seg_mean(sx1, medium_to_coarse, P2)
    pes2 = _seg_mean(pes1, medium_to_coarse, P2)
    bi0 = jnp.arange(B, dtype=jnp.int32) * 32
    bi1 = jnp.arange(B, dtype=jnp.int32) * 8
    bi2 = jnp.arange(B, dtype=jnp.int32) * 2
    ctx_idx = context_subgraph_idx + bi0
    tgt0 = target_subgraph_idxs + bi0[:, None]
    ctx_patch = sx0[ctx_idx] + jax.nn.relu(raw_patch_pes[ctx_idx] @ prw_Ws[0]
                                           + prw_bs[0])
    pe0 = jax.nn.relu(raw_patch_pes[tgt0.flatten()] @ prw_Ws[0]
                      + prw_bs[0]).reshape(B, NT0, D)
    cmask = mask[ctx_idx].astype(jnp.float32)[:, None, None]
    ctx_x0 = jax.nn.relu(ctx_patch[:, None, :] @ enc_Ws[0] + enc_bs[0]) * cmask
    tgt_x0 = sx0[tgt0.flatten()].reshape(B, NT0, D)
    tgt_x0 = jax.nn.relu(tgt_x0 @ enc_Ws[1] + enc_bs[1])
    pred0 = _mlp(ctx_x0 + pe0, pred00_Ws, pred00_bs, False)
    tgt1 = target_subgraph_idxs_L1 + bi1[:, None]
    pe1 = jax.nn.relu(pes1[tgt1.flatten()] @ prw_Ws[1]
                      + prw_bs[1]).reshape(B, NT1, D)
    tgt_x1 = sx1[tgt1.flatten()].reshape(B, NT1, D)
    tgt_x1 = jax.nn.relu(tgt_x1 @ enc_Ws[3] + enc_bs[3])
    ctx_x1 = jax.nn.relu(ctx_patch[:, None, :] @ enc_Ws[2] + enc_bs[2])
    pred1 = _mlp(ctx_x1 + pe1, pred01_Ws, pred01_bs, False)
    ctx_idx_L1 = fine_to_medium[ctx_idx]
    ctx_patch1 = sx1[ctx_idx_L1] + jax.nn.relu(pes1[ctx_idx_L1] @ prw_Ws[1]
                                               + prw_bs[1])
    tgt2 = target_subgraph_idxs_L2 + bi2[:, None]
    pe2 = jax.nn.relu(pes2[tgt2.flatten()] @ prw_Ws[2]
                      + prw_bs[2]).reshape(B, NT2, D)
    tgt_x2 = sx2[tgt2.flatten()].reshape(B, NT2, D)
    tgt_x2 = jax.nn.relu(tgt_x2 @ enc_Ws[5] + enc_bs[5])
    ctx_x2 = jax.nn.relu(ctx_patch1[:, None, :] @ enc_Ws[4] + enc_bs[4])
    pred2 = _mlp(ctx_x2 + pe2, pred12_Ws, pred12_bs, False)

    def mse(a, b):
        return jnp.mean((a - b) ** 2)

    def var_reg(p):
        std = jnp.sqrt(jnp.var(p.reshape(-1, D), axis=0) + 1e-4)
        return jnp.mean(jax.nn.relu(1.0 - std))

    loss = (1.0 * mse(pred0, tgt_x0) + 0.5 * mse(pred1, tgt_x1)
            + 0.25 * mse(pred2, tgt_x2))
    loss = loss + 0.01 * (var_reg(pred0) + var_reg(pred1) + var_reg(pred2))
    return loss


# speedup vs baseline: 1.9069x; 1.0317x over previous
import functools

import jax
import jax.numpy as jnp
from jax import lax
from jax.experimental import pallas as pl
from jax.experimental.pallas import tpu as pltpu
from jax.experimental.pallas import tpu_sc as plsc

N = 10000
E = 320000
D = 128
DE = 16
PRW = 16
B = 8
P0 = 256
P1 = 64
P2 = 16
NT0 = 4
NT1 = 4
NT2 = 1

NC = 2
NS = 16
NW = NC * NS
EC = 80
N_CHUNKS = E // EC
CHUNK_ITERS = -(-N_CHUNKS // NW)
ZROWS = 40
TILE_ROWS = 640


def _mp_body(h_hbm, e_hbm, src_hbm, dst_hbm, emap_hbm, out_hbm,
             src_v, dst_v, emap_v, dstS_v, hrow_v, erow_v, agg_sh,
             semL, semG, semE, semS):
    c = lax.axis_index("c")
    s = lax.axis_index("s")
    wid = s * NC + c
    n_copies = jnp.where(s < NS - 1, TILE_ROWS // ZROWS, 10)

    def zrow(i, carry):
        for g in range(8):
            hrow_v[0, i, pl.ds(g * 16, 16)] = jnp.zeros((16,), jnp.float32)
        return carry

    lax.fori_loop(0, ZROWS, zrow, 0)

    def zcp(j, carry):
        pltpu.sync_copy(hrow_v.at[0, pl.ds(0, ZROWS)],
                        agg_sh.at[pl.ds(s * TILE_ROWS + j * ZROWS, ZROWS)])
        return carry

    lax.fori_loop(0, n_copies, zcp, 0)
    plsc.subcore_barrier()

    def fire_loads(b, cid):
        base = cid * EC
        pltpu.async_copy(src_hbm.at[pl.ds(base, EC)], src_v.at[b], semL[b])
        pltpu.async_copy(dst_hbm.at[pl.ds(base, EC)], dst_v.at[b], semL[b])
        pltpu.async_copy(emap_hbm.at[pl.ds(base, EC)], emap_v.at[b], semL[b])

    for b in range(2):
        fire_loads(b, b * NW + wid)

    def round_for(b, cid):
        @pl.when(cid < N_CHUNKS)
        def _():
            @pl.when(cid >= 2 * NW)
            def _():
                pltpu.make_async_copy(hrow_v.at[b],
                                      agg_sh.at[dstS_v.at[b]], semS[b]).wait()

            base = cid * EC
            pltpu.make_async_copy(src_hbm.at[pl.ds(base, EC)], src_v.at[b],
                                  semL[b]).wait()
            pltpu.make_async_copy(dst_hbm.at[pl.ds(base, EC)], dst_v.at[b],
                                  semL[b]).wait()
            pltpu.make_async_copy(emap_hbm.at[pl.ds(base, EC)], emap_v.at[b],
                                  semL[b]).wait()
            gh = pltpu.async_copy(h_hbm.at[src_v.at[b]], hrow_v.at[b],
                                  semG[b])
            ge = pltpu.async_copy(e_hbm.at[emap_v.at[b]], erow_v.at[b],
                                  semE[b])
            gh.wait()
            ge.wait()
            for g in range(EC // 16):
                sl = pl.ds(g * 16, 16)
                dstS_v[b, sl] = dst_v[b, sl]

            def row(i, rc):
                for g in range(8):
                    sl = pl.ds(g * 16, 16)
                    hrow_v[b, i, sl] = jnp.maximum(
                        hrow_v[b, i, sl] + erow_v[b, i, sl], 0.0)
                return rc

            lax.fori_loop(0, EC, row, 0)
            pltpu.async_copy(hrow_v.at[b], agg_sh.at[dstS_v.at[b]], semS[b],
                             add=True)

            @pl.when(cid + 2 * NW < N_CHUNKS)
            def _():
                fire_loads(b, cid + 2 * NW)

    def round_pair(j2, carry):
        for b in range(2):
            round_for(b, (2 * j2 + b) * NW + wid)
        return carry

    lax.fori_loop(0, (CHUNK_ITERS + 1) // 2, round_pair, 0)
    for b in range(2):
        pltpu.make_async_copy(hrow_v.at[b], agg_sh.at[dstS_v.at[b]],
                              semS[b]).wait()
    plsc.subcore_barrier()

    def wout(j, carry):
        r0 = s * TILE_ROWS + j * ZROWS
        pltpu.sync_copy(agg_sh.at[pl.ds(r0, ZROWS)],
                        out_hbm.at[c, pl.ds(r0, ZROWS)])
        return carry

    lax.fori_loop(0, n_copies, wout, 0)


_mp_call = pl.kernel(
    _mp_body,
    out_type=jax.ShapeDtypeStruct((NC, N, D), jnp.float32),
    mesh=plsc.VectorSubcoreMesh(core_axis_name="c", subcore_axis_name="s"),
    scratch_types=[
        pltpu.VMEM((2, EC), jnp.int32),
        pltpu.VMEM((2, EC), jnp.int32),
        pltpu.VMEM((2, EC), jnp.int32),
        pltpu.VMEM((2, EC), jnp.int32),
        pltpu.VMEM((2, EC, D), jnp.float32),
        pltpu.VMEM((2, EC, D), jnp.float32),
        pltpu.VMEM_SHARED((N, D), jnp.float32),
        (pltpu.SemaphoreType.DMA, pltpu.SemaphoreType.DMA),
        (pltpu.SemaphoreType.DMA, pltpu.SemaphoreType.DMA),
        (pltpu.SemaphoreType.DMA, pltpu.SemaphoreType.DMA),
        (pltpu.SemaphoreType.DMA, pltpu.SemaphoreType.DMA),
    ],
)

N_GPAD = 10240
G_CHUNKS = N_GPAD // EC
G_ITERS = -(-G_CHUNKS // NW)


def _gather_body(table_hbm, idx_hbm, out_hbm, idx_v, rows_v, semG):
    c = lax.axis_index("c")
    s = lax.axis_index("s")
    wid = s * NC + c

    def chunk(j, carry):
        cid = j * NW + wid

        @pl.when(cid < G_CHUNKS)
        def _():
            base = cid * EC
            pltpu.sync_copy(idx_hbm.at[pl.ds(base, EC)], idx_v)
            pltpu.async_copy(table_hbm.at[idx_v], rows_v, semG).wait()
            pltpu.sync_copy(rows_v, out_hbm.at[pl.ds(base, EC)])

        return carry

    lax.fori_loop(0, G_ITERS, chunk, 0)


_gather128 = pl.kernel(
    _gather_body,
    out_type=jax.ShapeDtypeStruct((N_GPAD, D), jnp.float32),
    mesh=plsc.VectorSubcoreMesh(core_axis_name="c", subcore_axis_name="s"),
    scratch_types=[
        pltpu.VMEM((EC,), jnp.int32),
        pltpu.VMEM((EC, D), jnp.float32),
        pltpu.SemaphoreType.DMA,
    ],
)


def _me_kernel(x_ref, w_ref, b_ref, o_ref):
    acc = jnp.dot(x_ref[...], w_ref[...], preferred_element_type=jnp.float32)
    acc = jnp.maximum(acc + b_ref[...], 0.0)
    o_ref[...] = acc.reshape(o_ref.shape)


def _matmul_e(x, w, b, block_rows=800):
    r, k = x.shape
    n = w.shape[1]
    assert r % block_rows == 0
    return pl.pallas_call(
        _me_kernel,
        grid=(r // block_rows,),
        in_specs=[
            pl.BlockSpec((block_rows, k), lambda i: (i, 0)),
            pl.BlockSpec((k, n), lambda i: (0, 0)),
            pl.BlockSpec((n,), lambda i: (0,)),
        ],
        out_specs=pl.BlockSpec((block_rows * 8, n // 8), lambda i: (i, 0)),
        out_shape=jax.ShapeDtypeStruct((r * 8, n // 8), jnp.float32),
    )(x, w, b)


RC = 80
S_CHUNKS = N // RC
S_ITERS = -(-S_CHUNKS // NW)


def _ssn_body(x_hbm, idx_hbm, sum_hbm,
              idx_v, rows_v, sum_sh, semG):
    c = lax.axis_index("c")
    s = lax.axis_index("s")
    wid = s * NC + c
    n_copies = jnp.where(s < NS - 1, TILE_ROWS // ZROWS, 10)

    def zrow(i, carry):
        for g in range(8):
            rows_v[i, pl.ds(g * 16, 16)] = jnp.zeros((16,), jnp.float32)
        return carry

    lax.fori_loop(0, ZROWS, zrow, 0)

    def zcp(j, carry):
        r0 = s * TILE_ROWS + j * ZROWS
        pltpu.sync_copy(rows_v.at[pl.ds(0, ZROWS)],
                        sum_sh.at[pl.ds(r0, ZROWS)])
        return carry

    lax.fori_loop(0, n_copies, zcp, 0)
    plsc.subcore_barrier()

    def chunk(j, carry):
        cid = j * NW + wid

        @pl.when(cid < S_CHUNKS)
        def _():
            base = cid * RC
            pltpu.sync_copy(idx_hbm.at[pl.ds(base, RC)], idx_v)
            pltpu.sync_copy(x_hbm.at[pl.ds(base, RC)], rows_v)
            pltpu.sync_copy(rows_v, sum_sh.at[idx_v], add=True)

        return carry

    lax.fori_loop(0, S_ITERS, chunk, 0)
    plsc.subcore_barrier()

    def wout(j, carry):
        r0 = s * TILE_ROWS + j * ZROWS
        pltpu.sync_copy(sum_sh.at[pl.ds(r0, ZROWS)],
                        sum_hbm.at[c, pl.ds(r0, ZROWS)])
        return carry

    lax.fori_loop(0, n_copies, wout, 0)


_ssn_call = pl.kernel(
    _ssn_body,
    out_type=jax.ShapeDtypeStruct((NC, N, D), jnp.float32),
    mesh=plsc.VectorSubcoreMesh(core_axis_name="c", subcore_axis_name="s"),
    scratch_types=[
        pltpu.VMEM((RC,), jnp.int32),
        pltpu.VMEM((RC, D), jnp.float32),
        pltpu.VMEM_SHARED((N, D), jnp.float32),
        pltpu.SemaphoreType.DMA,
    ],
)


def _ss256_kernel(idx_ref, x_ref, sum_ref, cnt_ref):
    i = pl.program_id(0)
    idx = idx_ref[0, 0, :]
    oh = (idx[:, None]
          == lax.broadcasted_iota(jnp.int32, (idx.shape[0], P0), 1)
          ).astype(jnp.float32)
    part = lax.dot_general(oh, x_ref[...], (((0,), (0,)), ((), ())),
                           preferred_element_type=jnp.float32)
    pc = jnp.sum(oh, axis=0)

    @pl.when(i == 0)
    def _():
        sum_ref[...] = part
        cnt_ref[...] = pc

    @pl.when(i > 0)
    def _():
        sum_ref[...] = sum_ref[...] + part
        cnt_ref[...] = cnt_ref[...] + pc


def _segsum256(data, idx, block_rows=2000):
    r = data.shape[0]
    assert r % block_rows == 0
    return pl.pallas_call(
        _ss256_kernel,
        grid=(r // block_rows,),
        in_specs=[
            pl.BlockSpec((1, 1, block_rows), lambda i: (i, 0, 0)),
            pl.BlockSpec((block_rows, D), lambda i: (i, 0)),
        ],
        out_specs=(pl.BlockSpec((P0, D), lambda i: (0, 0)),
                   pl.BlockSpec((P0,), lambda i: (0,))),
        out_shape=(jax.ShapeDtypeStruct((P0, D), jnp.float32),
                   jax.ShapeDtypeStruct((P0,), jnp.float32)),
    )(idx.reshape(r // block_rows, 1, block_rows), data)

def _mm_kernel(x_ref, w_ref, b_ref, o_ref, *, relu):
    acc = jnp.dot(x_ref[...], w_ref[...], preferred_element_type=jnp.float32)
    acc = acc + b_ref[...]
    if relu:
        acc = jnp.maximum(acc, 0.0)
    o_ref[...] = acc


def _matmul(x, w, b, relu=True, block_rows=400):
    r, k = x.shape
    n = w.shape[1]
    assert r % block_rows == 0, (r, block_rows)
    out = pl.pallas_call(
        functools.partial(_mm_kernel, relu=relu),
        grid=(r // block_rows,),
        in_specs=[
            pl.BlockSpec((block_rows, k), lambda i: (i, 0)),
            pl.BlockSpec((k, n), lambda i: (0, 0)),
            pl.BlockSpec((n,), lambda i: (0,)),
        ],
        out_specs=pl.BlockSpec((block_rows, n), lambda i: (i, 0)),
        out_shape=jax.ShapeDtypeStruct((r, n), jnp.float32),
    )(x, w, b)
    return out


def _mm3_kernel(x_ref, a0_ref, a1_ref, w_ref, b_ref, o_ref):
    acc = x_ref[...] + a0_ref[...] + a1_ref[...]
    acc = jnp.dot(acc, w_ref[...], preferred_element_type=jnp.float32)
    o_ref[...] = jnp.maximum(acc + b_ref[...], 0.0)


def _mm3(x, a0, a1, w, b, block_rows=400):
    r, k = x.shape
    n = w.shape[1]
    assert r % block_rows == 0
    return pl.pallas_call(
        _mm3_kernel,
        grid=(r // block_rows,),
        in_specs=[
            pl.BlockSpec((block_rows, k), lambda i: (i, 0)),
            pl.BlockSpec((block_rows, k), lambda i: (i, 0)),
            pl.BlockSpec((block_rows, k), lambda i: (i, 0)),
            pl.BlockSpec((k, n), lambda i: (0, 0)),
            pl.BlockSpec((n,), lambda i: (0,)),
        ],
        out_specs=pl.BlockSpec((block_rows, n), lambda i: (i, 0)),
        out_shape=jax.ShapeDtypeStruct((r, n), jnp.float32),
    )(x, a0, a1, w, b)


def _seg_mean(data, seg, num):
    s = jax.ops.segment_sum(data, seg, num_segments=num)
    c = jax.ops.segment_sum(jnp.ones((data.shape[0], 1), data.dtype), seg,
                            num_segments=num)
    return s / jnp.maximum(c, 1.0)


def _mlp(h, Ws, bs, final_act):
    n = Ws.shape[0]
    for i in range(n):
        h = h @ Ws[i] + bs[i]
        if i < n - 1 or final_act:
            h = jax.nn.relu(h)
    return h


def kernel(x, edge_attr, rw_pos_enc, W_in, b_in, W_edge, b_edge, gnn_Ws,
           gnn_bs, U_W, U_b, prw_Ws, prw_bs, enc_Ws, enc_bs, pred00_Ws,
           pred00_bs, pred01_Ws, pred01_bs, pred12_Ws, pred12_bs, edge_index,
           subgraphs_nodes_mapper, subgraphs_edges_mapper, subgraphs_batch,
           fine_to_medium, medium_to_coarse, context_subgraph_idx,
           target_subgraph_idxs, target_subgraph_idxs_L1,
           target_subgraph_idxs_L2, mask):
    src, dst = edge_index[0], edge_index[1]
    map_pad = jnp.concatenate(
        [subgraphs_nodes_mapper, jnp.zeros((N_GPAD - N,), jnp.int32)])

    h = _gather128(_matmul(x, W_in, b_in), map_pad)[:N]
    W_blk = jnp.kron(jnp.eye(8, dtype=jnp.float32), W_edge)
    b_blk = jnp.tile(b_edge, 8)
    e = _matmul_e(edge_attr.reshape(E // 8, 8 * DE), W_blk, b_blk,
                  block_rows=800)

    pes = rw_pos_enc[subgraphs_nodes_mapper]
    raw_patch_pes = jax.ops.segment_max(pes, subgraphs_batch, num_segments=P0)

    agg = _mp_call(h, e, src, dst, subgraphs_edges_mapper)
    h = _mm3(h, agg[0], agg[1], gnn_Ws[0], gnn_bs[0])

    batch_pad = jnp.concatenate(
        [subgraphs_batch, jnp.zeros((N_GPAD - N,), jnp.int32)])
    bsum, bcnt = _segsum256(h, subgraphs_batch)
    t = _matmul(bsum / jnp.maximum(bcnt, 1.0)[:, None], U_W, U_b,
                block_rows=P0)
    h = h + _gather128(t, batch_pad)[:N]
    nsum = _ssn_call(h, subgraphs_nodes_mapper)
    ncnt = jax.ops.segment_sum(jnp.ones((N, 1), jnp.float32),
                               subgraphs_nodes_mapper, num_segments=N)
    node_mean = (nsum[0] + nsum[1]) / jnp.maximum(ncnt, 1.0)
    h = _gather128(node_mean, map_pad)[:N]

    agg = _mp_call(h, e, src, dst, subgraphs_edges_mapper)
    h = _mm3(h, agg[0], agg[1], gnn_Ws[1], gnn_bs[1])

    s0sum, s0cnt = _segsum256(h, subgraphs_batch)
    sx0 = s0sum / jnp.maximum(s0cnt, 1.0)[:, None]
    sx1 = _seg_mean(sx0, fine_to_medium, P1)
    pes1 = _seg_mean(raw_patch_pes, fine_to_medium, P1)
    sx2 = _seg_mean(sx1, medium_to_coarse, P2)
    pes2 = _seg_mean(pes1, medium_to_coarse, P2)
    bi0 = jnp.arange(B, dtype=jnp.int32) * 32
    bi1 = jnp.arange(B, dtype=jnp.int32) * 8
    bi2 = jnp.arange(B, dtype=jnp.int32) * 2
    ctx_idx = context_subgraph_idx + bi0
    tgt0 = target_subgraph_idxs + bi0[:, None]
    ctx_patch = sx0[ctx_idx] + jax.nn.relu(raw_patch_pes[ctx_idx] @ prw_Ws[0]
                                           + prw_bs[0])
    pe0 = jax.nn.relu(raw_patch_pes[tgt0.flatten()] @ prw_Ws[0]
                      + prw_bs[0]).reshape(B, NT0, D)
    cmask = mask[ctx_idx].astype(jnp.float32)[:, None, None]
    ctx_x0 = jax.nn.relu(ctx_patch[:, None, :] @ enc_Ws[0] + enc_bs[0]) * cmask
    tgt_x0 = sx0[tgt0.flatten()].reshape(B, NT0, D)
    tgt_x0 = jax.nn.relu(tgt_x0 @ enc_Ws[1] + enc_bs[1])
    pred0 = _mlp(ctx_x0 + pe0, pred00_Ws, pred00_bs, False)
    tgt1 = target_subgraph_idxs_L1 + bi1[:, None]
    pe1 = jax.nn.relu(pes1[tgt1.flatten()] @ prw_Ws[1]
                      + prw_bs[1]).reshape(B, NT1, D)
    tgt_x1 = sx1[tgt1.flatten()].reshape(B, NT1, D)
    tgt_x1 = jax.nn.relu(tgt_x1 @ enc_Ws[3] + enc_bs[3])
    ctx_x1 = jax.nn.relu(ctx_patch[:, None, :] @ enc_Ws[2] + enc_bs[2])
    pred1 = _mlp(ctx_x1 + pe1, pred01_Ws, pred01_bs, False)
    ctx_idx_L1 = fine_to_medium[ctx_idx]
    ctx_patch1 = sx1[ctx_idx_L1] + jax.nn.relu(pes1[ctx_idx_L1] @ prw_Ws[1]
                                               + prw_bs[1])
    tgt2 = target_subgraph_idxs_L2 + bi2[:, None]
    pe2 = jax.nn.relu(pes2[tgt2.flatten()] @ prw_Ws[2]
                      + prw_bs[2]).reshape(B, NT2, D)
    tgt_x2 = sx2[tgt2.flatten()].reshape(B, NT2, D)
    tgt_x2 = jax.nn.relu(tgt_x2 @ enc_Ws[5] + enc_bs[5])
    ctx_x2 = jax.nn.relu(ctx_patch1[:, None, :] @ enc_Ws[4] + enc_bs[4])
    pred2 = _mlp(ctx_x2 + pe2, pred12_Ws, pred12_bs, False)

    def mse(a, b):
        return jnp.mean((a - b) ** 2)

    def var_reg(p):
        std = jnp.sqrt(jnp.var(p.reshape(-1, D), axis=0) + 1e-4)
        return jnp.mean(jax.nn.relu(1.0 - std))

    loss = (1.0 * mse(pred0, tgt_x0) + 0.5 * mse(pred1, tgt_x1)
            + 0.25 * mse(pred2, tgt_x2))
    loss = loss + 0.01 * (var_reg(pred0) + var_reg(pred1) + var_reg(pred2))
    return loss
